# baseline (speedup 1.0000x reference)
import functools

import jax
import jax.numpy as jnp
from jax import lax
from jax.experimental import pallas as pl
from jax.experimental.pallas import tpu as pltpu
from jax.experimental.pallas import tpu_sc as plsc

N_NODES_P = 10240
ROWS_PER_TILE = 640
E_CHUNKS = 2500
CH_MAIN = 78
CH_EXTRA_W = 4
CH_MAX = 79
NBUF = 2
CHUNK = 128
NC, NS = 2, 16
F = 64
DEG_W = 16


def _sc_mesh():
    return plsc.VectorSubcoreMesh(core_axis_name="c", subcore_axis_name="s",
                                  num_cores=NC, num_subcores=NS)


def _make_degree_kernel():
    mesh = _sc_mesh()

    @functools.partial(
        pl.kernel,
        out_type=jax.ShapeDtypeStruct((NC, N_NODES_P, DEG_W), jnp.float32),
        mesh=mesh,
        compiler_params=pltpu.CompilerParams(use_tc_tiling_on_sc=False),
        scratch_types=[
            pltpu.VMEM((CH_MAX, CHUNK), jnp.int32),
            pltpu.VMEM((CHUNK, DEG_W), jnp.float32),
            pltpu.VMEM((ROWS_PER_TILE, DEG_W), jnp.float32),
            pltpu.VMEM_SHARED((N_NODES_P, DEG_W), jnp.float32),
            pltpu.SemaphoreType.DMA,
        ],
    )
    def deg_kernel(ei_hbm, out_hbm, dst_v, ones_v, zbuf, acc_sh, sem):
        cid = lax.axis_index("c")
        sid = lax.axis_index("s")
        wid = sid * NC + cid
        c0 = wid * CH_MAIN + jnp.minimum(wid, CH_EXTRA_W)
        extra = wid < CH_EXTRA_W

        one_row = jnp.where(lax.iota(jnp.int32, 16) == 0, 1.0, 0.0).astype(jnp.float32)
        zero = jnp.zeros((16,), jnp.float32)

        def fill(i, _):
            ones_v[i, :] = one_row
            return 0
        lax.fori_loop(0, CHUNK, fill, 0)

        def zfill(i, _):
            zbuf[i, :] = zero
            return 0
        lax.fori_loop(0, ROWS_PER_TILE, zfill, 0)

        pltpu.sync_copy(zbuf, acc_sh.at[pl.ds(sid * ROWS_PER_TILE, ROWS_PER_TILE)])
        plsc.subcore_barrier()

        pltpu.sync_copy(ei_hbm.at[1, pl.ds(c0, CH_MAIN)],
                        dst_v.at[pl.ds(0, CH_MAIN)])

        @pl.when(extra)
        def _():
            pltpu.sync_copy(ei_hbm.at[1, pl.ds(c0 + CH_MAIN, 1)],
                            dst_v.at[pl.ds(CH_MAIN, 1)])

        def chunk(j, _):
            pltpu.sync_copy(ones_v, acc_sh.at[dst_v.at[j]], add=True)
            return 0
        nw = jnp.where(extra, CH_MAIN + 1, CH_MAIN)
        lax.fori_loop(0, nw, chunk, 0)

        plsc.subcore_barrier()
        pltpu.sync_copy(
            acc_sh.at[pl.ds(sid * ROWS_PER_TILE, ROWS_PER_TILE)],
            out_hbm.at[cid, pl.ds(sid * ROWS_PER_TILE, ROWS_PER_TILE)])

    return deg_kernel


def _make_aggregate_kernel():
    mesh = _sc_mesh()

    @functools.partial(
        pl.kernel,
        out_type=jax.ShapeDtypeStruct((NC, N_NODES_P, F), jnp.float32),
        mesh=mesh,
        compiler_params=pltpu.CompilerParams(use_tc_tiling_on_sc=False),
        scratch_types=[
            pltpu.VMEM((CH_MAX, CHUNK), jnp.int32),
            pltpu.VMEM((CH_MAX, CHUNK), jnp.int32),
            pltpu.VMEM((NBUF, CHUNK, F), jnp.float32),
            pltpu.VMEM((CHUNK, F), jnp.float32),
            pltpu.VMEM_SHARED((N_NODES_P, F), jnp.float32),
            pltpu.VMEM_SHARED((N_NODES_P, F), jnp.float32),
        ] + [pltpu.SemaphoreType.DMA] * (2 * NBUF),
    )
    def agg_kernel(ei_hbm, y_hbm, out_hbm,
                   src_v, dst_v, rows_v, zbuf, acc_sh, y_sp, *sems):
        gsems = sems[:NBUF]
        ssems = sems[NBUF:]
        cid = lax.axis_index("c")
        sid = lax.axis_index("s")
        wid = sid * NC + cid
        c0 = wid * CH_MAIN + jnp.minimum(wid, CH_EXTRA_W)
        extra = wid < CH_EXTRA_W
        nw = jnp.where(extra, CH_MAIN + 1, CH_MAIN)

        zero = jnp.zeros((16,), jnp.float32)

        def zfill(i, _):
            for j in range(F // 16):
                zbuf[i, pl.ds(j * 16, 16)] = zero
            return 0
        lax.fori_loop(0, CHUNK, zfill, 0)

        pltpu.sync_copy(y_hbm.at[pl.ds(sid * ROWS_PER_TILE, ROWS_PER_TILE)],
                        y_sp.at[pl.ds(sid * ROWS_PER_TILE, ROWS_PER_TILE)])

        for k in range(ROWS_PER_TILE // CHUNK):
            pltpu.sync_copy(
                zbuf, acc_sh.at[pl.ds(sid * ROWS_PER_TILE + k * CHUNK, CHUNK)])
        plsc.subcore_barrier()

        pltpu.sync_copy(ei_hbm.at[0, pl.ds(c0, CH_MAIN)],
                        src_v.at[pl.ds(0, CH_MAIN)])
        pltpu.sync_copy(ei_hbm.at[1, pl.ds(c0, CH_MAIN)],
                        dst_v.at[pl.ds(0, CH_MAIN)])

        @pl.when(extra)
        def _():
            pltpu.sync_copy(ei_hbm.at[0, pl.ds(c0 + CH_MAIN, 1)],
                            src_v.at[pl.ds(CH_MAIN, 1)])
            pltpu.sync_copy(ei_hbm.at[1, pl.ds(c0 + CH_MAIN, 1)],
                            dst_v.at[pl.ds(CH_MAIN, 1)])

        for b in range(NBUF):
            pltpu.async_copy(y_sp.at[src_v.at[b]], rows_v.at[b], gsems[b])

        def group(g, _):
            for b in range(NBUF):
                j = g * NBUF + b
                bp = (b - 1) % NBUF
                pltpu.make_async_copy(
                    y_sp.at[src_v.at[j]], rows_v.at[b], gsems[b]).wait()
                pltpu.async_copy(rows_v.at[b], acc_sh.at[dst_v.at[j]],
                                 ssems[b], add=True)

                @pl.when(j >= 1)
                def _():
                    pltpu.make_async_copy(
                        rows_v.at[bp], acc_sh.at[dst_v.at[j - 1]],
                        ssems[bp]).wait()

                @pl.when((j >= 1) & (j - 1 + NBUF < nw))
                def _():
                    pltpu.async_copy(
                        y_sp.at[src_v.at[j - 1 + NBUF]], rows_v.at[bp],
                        gsems[bp])
            return 0
        lax.fori_loop(0, CH_MAIN // NBUF, group, 0)

        bl = (CH_MAIN - 1) % NBUF

        @pl.when(extra)
        def _():
            b = CH_MAIN % NBUF
            pltpu.make_async_copy(
                y_sp.at[src_v.at[CH_MAIN]], rows_v.at[b], gsems[b]).wait()
            pltpu.async_copy(rows_v.at[b], acc_sh.at[dst_v.at[CH_MAIN]],
                             ssems[b], add=True)
            pltpu.make_async_copy(
                rows_v.at[b], acc_sh.at[dst_v.at[CH_MAIN]], ssems[b]).wait()

        pltpu.make_async_copy(
            rows_v.at[bl], acc_sh.at[dst_v.at[CH_MAIN - 1]], ssems[bl]).wait()

        plsc.subcore_barrier()
        pltpu.sync_copy(
            acc_sh.at[pl.ds(sid * ROWS_PER_TILE, ROWS_PER_TILE)],
            out_hbm.at[cid, pl.ds(sid * ROWS_PER_TILE, ROWS_PER_TILE)])

    return agg_kernel


NP2 = N_NODES_P // 2
NREAL2 = 5000


def _xw1_body(x_ref, w_ref, xw_ref):
    xw_ref[...] = jnp.dot(x_ref[...], w_ref[...],
                          preferred_element_type=jnp.float32)


def _tc_xw1(x2, Wb1):
    return pl.pallas_call(
        _xw1_body,
        out_shape=jax.ShapeDtypeStruct((NREAL2, 128), jnp.float32),
    )(x2, Wb1)


def _scale1_body(xw_ref, dp_ref, y_ref, dis_ref):
    degw = dp_ref[0] + dp_ref[1]
    G = jnp.where(degw > 0, lax.rsqrt(jnp.maximum(degw, 1e-30)), 0.0)
    G4 = jnp.broadcast_to(G[:, None, :], (N_NODES_P // 8, 4, 128)
                          ).reshape(NP2, 128)
    lane = lax.broadcasted_iota(jnp.int32, (NP2, 128), 1)
    m = lax.broadcasted_iota(jnp.int32, (NP2, 128), 0) % 4
    ev = jnp.sum(jnp.where(lane == 32 * m, G4, 0.0), axis=1, keepdims=True)
    od = jnp.sum(jnp.where(lane == 32 * m + 16, G4, 0.0), axis=1,
                 keepdims=True)
    dis2 = jnp.concatenate(
        [jnp.broadcast_to(ev, (NP2, F)),
         jnp.broadcast_to(od, (NP2, F))], axis=1)
    dis_ref[...] = dis2
    y_ref[...] = jnp.concatenate(
        [dis2[:NREAL2, :] * xw_ref[...],
         jnp.zeros((NP2 - NREAL2, 128), jnp.float32)], axis=0)


def _tc_scale1(xw, dp):
    return pl.pallas_call(
        _scale1_body,
        out_shape=[jax.ShapeDtypeStruct((NP2, 128), jnp.float32),
                   jax.ShapeDtypeStruct((NP2, 128), jnp.float32)],
    )(xw, dp)


def _mid_body(agg_ref, dis_ref, b_ref, w_ref, xk_ref, y_ref):
    agg = agg_ref[0] + agg_ref[1]
    dis = dis_ref[...]
    xk = jnp.maximum(dis * agg + b_ref[...], 0.0)
    xk_ref[...] = xk
    y_ref[...] = dis * jnp.dot(xk, w_ref[...], preferred_element_type=jnp.float32)


def _tc_mid(agg_p, dis2, bb, Wb_next):
    return pl.pallas_call(
        _mid_body,
        out_shape=[jax.ShapeDtypeStruct((NP2, 128), jnp.float32),
                   jax.ShapeDtypeStruct((NP2, 128), jnp.float32)],
    )(agg_p, dis2, bb, Wb_next)


def _final_body(agg_ref, dis_ref, b_ref, x1_ref, x2_ref, pp_ref,
                wf_ref, bf_ref, out_ref):
    agg = agg_ref[0] + agg_ref[1]
    x3 = jnp.maximum(dis_ref[...] * agg + b_ref[...], 0.0)

    Pp = pp_ref[...]
    Pe = Pp[:, :64]
    Po = Pp[:, 64:]

    dn = (((0,), (0,)), ((), ()))

    def pool(xp):
        return (lax.dot_general(Pe, xp[:, :F], dn,
                                preferred_element_type=jnp.float32)
                + lax.dot_general(Po, xp[:, F:], dn,
                                  preferred_element_type=jnp.float32))

    s1 = pool(x1_ref[...])
    s2 = pool(x2_ref[...])
    s3 = pool(x3)
    pooled = jnp.concatenate([s1, s2, s3], axis=1)

    counts = jnp.sum(Pe, axis=0, keepdims=True) + jnp.sum(Po, axis=0,
                                                          keepdims=True)
    inv = 1.0 / jnp.maximum(counts, 1.0)
    pooled = pooled * inv.T

    logits = jnp.dot(pooled, wf_ref[...], preferred_element_type=jnp.float32)
    logits = logits + bf_ref[...]
    m = jnp.max(logits, axis=1, keepdims=True)
    e = jnp.exp(logits - m)
    out_ref[...] = e / jnp.sum(e, axis=1, keepdims=True)


def _tc_final(agg_p, dis2, bb3, x1p, x2p, Pp, Wf, bfr):
    return pl.pallas_call(
        _final_body,
        out_shape=jax.ShapeDtypeStruct((64, 10), jnp.float32),
    )(agg_p, dis2, bb3, x1p, x2p, Pp, Wf, bfr)


def _block_diag2(W):
    a, b = W.shape
    Z = jnp.zeros((a, b), W.dtype)
    return jnp.concatenate(
        [jnp.concatenate([W, Z], axis=1), jnp.concatenate([Z, W], axis=1)],
        axis=0)


def kernel(x, edge_index, batch, W1, b1, W2, b2, W3, b3, Wf, bf):
    n = x.shape[0]
    ei3 = edge_index.astype(jnp.int32).reshape(2, E_CHUNKS, CHUNK)

    x2 = x.reshape(NREAL2, 2 * x.shape[1])
    batch_pad = jnp.concatenate(
        [batch.astype(jnp.int32), jnp.full((N_NODES_P - n,), 64, jnp.int32)])
    Pp = (batch_pad[:, None] == jnp.arange(64, dtype=jnp.int32)[None, :]
          ).astype(jnp.float32).reshape(NP2, 128)

    deg_kernel = _make_degree_kernel()
    agg_kernel = _make_aggregate_kernel()

    dp = deg_kernel(ei3).reshape(NC, N_NODES_P * DEG_W // 128, 128)

    Wb1 = _block_diag2(W1)
    Wb2 = _block_diag2(W2)
    Wb3 = _block_diag2(W3)
    bb1 = jnp.concatenate([b1, b1]).reshape(1, 128)
    bb2 = jnp.concatenate([b2, b2]).reshape(1, 128)
    bb3 = jnp.concatenate([b3, b3]).reshape(1, 128)
    bfr = bf.reshape(1, 10)

    def agg(yp):
        parts = agg_kernel(ei3, yp.reshape(N_NODES_P, F))
        return parts.reshape(NC, NP2, 128)

    xw1 = _tc_xw1(x2, Wb1)
    y1p, dis2 = _tc_scale1(xw1, dp)
    x1p, y2p = _tc_mid(agg(y1p), dis2, bb1, Wb2)
    x2p, y3p = _tc_mid(agg(y2p), dis2, bb2, Wb3)
    return _tc_final(agg(y3p), dis2, bb3, x1p, x2p, Pp, Wf, bfr)


# speedup vs baseline: 1.3302x; 1.3302x over previous
import functools

import jax
import jax.numpy as jnp
from jax import lax
from jax.experimental import pallas as pl
from jax.experimental.pallas import tpu as pltpu
from jax.experimental.pallas import tpu_sc as plsc

N_NODES_P = 10240
ROWS_PER_TILE = 640
E_CHUNKS = 2500
CH_MAIN = 78
CH_EXTRA_W = 4
CH_MAX = 79
NBUF = 6
CHUNK = 128
NC, NS = 2, 16
F = 64
DEG_W = 16


def _sc_mesh():
    return plsc.VectorSubcoreMesh(core_axis_name="c", subcore_axis_name="s",
                                  num_cores=NC, num_subcores=NS)


def _make_degree_kernel():
    mesh = _sc_mesh()

    @functools.partial(
        pl.kernel,
        out_type=jax.ShapeDtypeStruct((NC, N_NODES_P, DEG_W), jnp.float32),
        mesh=mesh,
        compiler_params=pltpu.CompilerParams(use_tc_tiling_on_sc=False),
        scratch_types=[
            pltpu.VMEM((CH_MAX, CHUNK), jnp.int32),
            pltpu.VMEM((CHUNK, DEG_W), jnp.float32),
            pltpu.VMEM((ROWS_PER_TILE, DEG_W), jnp.float32),
            pltpu.VMEM_SHARED((N_NODES_P, DEG_W), jnp.float32),
            pltpu.SemaphoreType.DMA,
        ],
    )
    def deg_kernel(ei_hbm, out_hbm, dst_v, ones_v, zbuf, acc_sh, sem):
        cid = lax.axis_index("c")
        sid = lax.axis_index("s")
        wid = sid * NC + cid
        c0 = wid * CH_MAIN + jnp.minimum(wid, CH_EXTRA_W)
        extra = wid < CH_EXTRA_W

        one_row = jnp.where(lax.iota(jnp.int32, 16) == 0, 1.0, 0.0).astype(jnp.float32)
        zero = jnp.zeros((16,), jnp.float32)

        def fill(i, _):
            ones_v[i, :] = one_row
            return 0
        lax.fori_loop(0, CHUNK, fill, 0)

        def zfill(i, _):
            zbuf[i, :] = zero
            return 0
        lax.fori_loop(0, ROWS_PER_TILE, zfill, 0)

        pltpu.sync_copy(zbuf, acc_sh.at[pl.ds(sid * ROWS_PER_TILE, ROWS_PER_TILE)])
        plsc.subcore_barrier()

        pltpu.sync_copy(ei_hbm.at[1, pl.ds(c0, CH_MAIN)],
                        dst_v.at[pl.ds(0, CH_MAIN)])

        @pl.when(extra)
        def _():
            pltpu.sync_copy(ei_hbm.at[1, pl.ds(c0 + CH_MAIN, 1)],
                            dst_v.at[pl.ds(CH_MAIN, 1)])

        def chunk(j, _):
            pltpu.sync_copy(ones_v, acc_sh.at[dst_v.at[j]], add=True)
            return 0
        nw = jnp.where(extra, CH_MAIN + 1, CH_MAIN)
        lax.fori_loop(0, nw, chunk, 0)

        plsc.subcore_barrier()
        pltpu.sync_copy(
            acc_sh.at[pl.ds(sid * ROWS_PER_TILE, ROWS_PER_TILE)],
            out_hbm.at[cid, pl.ds(sid * ROWS_PER_TILE, ROWS_PER_TILE)])

    return deg_kernel


def _make_aggregate_kernel():
    mesh = _sc_mesh()

    @functools.partial(
        pl.kernel,
        out_type=jax.ShapeDtypeStruct((NC, N_NODES_P, F), jnp.float32),
        mesh=mesh,
        compiler_params=pltpu.CompilerParams(use_tc_tiling_on_sc=False),
        scratch_types=[
            pltpu.VMEM((CH_MAX, CHUNK), jnp.int32),
            pltpu.VMEM((CH_MAX, CHUNK), jnp.int32),
            pltpu.VMEM((NBUF, CHUNK, F), jnp.float32),
            pltpu.VMEM((CHUNK, F), jnp.float32),
            pltpu.VMEM_SHARED((N_NODES_P, F), jnp.float32),
        ] + [pltpu.SemaphoreType.DMA] * (2 * NBUF),
    )
    def agg_kernel(ei_hbm, y_hbm, out_hbm,
                   src_v, dst_v, rows_v, zbuf, acc_sh, *sems):
        gsems = sems[:NBUF]
        ssems = sems[NBUF:]
        cid = lax.axis_index("c")
        sid = lax.axis_index("s")
        wid = sid * NC + cid
        c0 = wid * CH_MAIN + jnp.minimum(wid, CH_EXTRA_W)
        extra = wid < CH_EXTRA_W
        nw = jnp.where(extra, CH_MAIN + 1, CH_MAIN)

        zero = jnp.zeros((16,), jnp.float32)

        def zfill(i, _):
            for j in range(F // 16):
                zbuf[i, pl.ds(j * 16, 16)] = zero
            return 0
        lax.fori_loop(0, CHUNK, zfill, 0)

        for k in range(ROWS_PER_TILE // CHUNK):
            pltpu.sync_copy(
                zbuf, acc_sh.at[pl.ds(sid * ROWS_PER_TILE + k * CHUNK, CHUNK)])
        plsc.subcore_barrier()

        pltpu.sync_copy(ei_hbm.at[0, pl.ds(c0, CH_MAIN)],
                        src_v.at[pl.ds(0, CH_MAIN)])
        pltpu.sync_copy(ei_hbm.at[1, pl.ds(c0, CH_MAIN)],
                        dst_v.at[pl.ds(0, CH_MAIN)])

        @pl.when(extra)
        def _():
            pltpu.sync_copy(ei_hbm.at[0, pl.ds(c0 + CH_MAIN, 1)],
                            src_v.at[pl.ds(CH_MAIN, 1)])
            pltpu.sync_copy(ei_hbm.at[1, pl.ds(c0 + CH_MAIN, 1)],
                            dst_v.at[pl.ds(CH_MAIN, 1)])

        for b in range(NBUF):
            pltpu.async_copy(y_hbm.at[src_v.at[b]], rows_v.at[b], gsems[b])

        def group(g, _):
            for b in range(NBUF):
                j = g * NBUF + b
                bp = (b - 1) % NBUF
                pltpu.make_async_copy(
                    y_hbm.at[src_v.at[j]], rows_v.at[b], gsems[b]).wait()
                pltpu.async_copy(rows_v.at[b], acc_sh.at[dst_v.at[j]],
                                 ssems[b], add=True)

                @pl.when(j >= 1)
                def _():
                    pltpu.make_async_copy(
                        rows_v.at[bp], acc_sh.at[dst_v.at[j - 1]],
                        ssems[bp]).wait()

                @pl.when((j >= 1) & (j - 1 + NBUF < nw))
                def _():
                    pltpu.async_copy(
                        y_hbm.at[src_v.at[j - 1 + NBUF]], rows_v.at[bp],
                        gsems[bp])
            return 0
        lax.fori_loop(0, CH_MAIN // NBUF, group, 0)

        bl = (CH_MAIN - 1) % NBUF

        @pl.when(extra)
        def _():
            b = CH_MAIN % NBUF
            pltpu.make_async_copy(
                y_hbm.at[src_v.at[CH_MAIN]], rows_v.at[b], gsems[b]).wait()
            pltpu.async_copy(rows_v.at[b], acc_sh.at[dst_v.at[CH_MAIN]],
                             ssems[b], add=True)
            pltpu.make_async_copy(
                rows_v.at[b], acc_sh.at[dst_v.at[CH_MAIN]], ssems[b]).wait()

        pltpu.make_async_copy(
            rows_v.at[bl], acc_sh.at[dst_v.at[CH_MAIN - 1]], ssems[bl]).wait()

        plsc.subcore_barrier()
        pltpu.sync_copy(
            acc_sh.at[pl.ds(sid * ROWS_PER_TILE, ROWS_PER_TILE)],
            out_hbm.at[cid, pl.ds(sid * ROWS_PER_TILE, ROWS_PER_TILE)])

    return agg_kernel


NP2 = N_NODES_P // 2
NREAL2 = 5000


def _xw1_body(x_ref, w_ref, xw_ref):
    xw_ref[...] = jnp.dot(x_ref[...], w_ref[...],
                          preferred_element_type=jnp.float32)


def _tc_xw1(x2, Wb1):
    return pl.pallas_call(
        _xw1_body,
        out_shape=jax.ShapeDtypeStruct((NREAL2, 128), jnp.float32),
    )(x2, Wb1)


def _scale1_body(xw_ref, dp_ref, y_ref, dis_ref):
    degw = dp_ref[0] + dp_ref[1]
    G = jnp.where(degw > 0, lax.rsqrt(jnp.maximum(degw, 1e-30)), 0.0)
    G4 = jnp.broadcast_to(G[:, None, :], (N_NODES_P // 8, 4, 128)
                          ).reshape(NP2, 128)
    lane = lax.broadcasted_iota(jnp.int32, (NP2, 128), 1)
    m = lax.broadcasted_iota(jnp.int32, (NP2, 128), 0) % 4
    ev = jnp.sum(jnp.where(lane == 32 * m, G4, 0.0), axis=1, keepdims=True)
    od = jnp.sum(jnp.where(lane == 32 * m + 16, G4, 0.0), axis=1,
                 keepdims=True)
    dis2 = jnp.concatenate(
        [jnp.broadcast_to(ev, (NP2, F)),
         jnp.broadcast_to(od, (NP2, F))], axis=1)
    dis_ref[...] = dis2
    y_ref[...] = jnp.concatenate(
        [dis2[:NREAL2, :] * xw_ref[...],
         jnp.zeros((NP2 - NREAL2, 128), jnp.float32)], axis=0)


def _tc_scale1(xw, dp):
    return pl.pallas_call(
        _scale1_body,
        out_shape=[jax.ShapeDtypeStruct((NP2, 128), jnp.float32),
                   jax.ShapeDtypeStruct((NP2, 128), jnp.float32)],
    )(xw, dp)


def _mid_body(agg_ref, dis_ref, b_ref, w_ref, xk_ref, y_ref):
    agg = agg_ref[0] + agg_ref[1]
    dis = dis_ref[...]
    xk = jnp.maximum(dis * agg + b_ref[...], 0.0)
    xk_ref[...] = xk
    y_ref[...] = dis * jnp.dot(xk, w_ref[...], preferred_element_type=jnp.float32)


def _tc_mid(agg_p, dis2, bb, Wb_next):
    return pl.pallas_call(
        _mid_body,
        out_shape=[jax.ShapeDtypeStruct((NP2, 128), jnp.float32),
                   jax.ShapeDtypeStruct((NP2, 128), jnp.float32)],
    )(agg_p, dis2, bb, Wb_next)


def _final_body(agg_ref, dis_ref, b_ref, x1_ref, x2_ref, pp_ref,
                wf_ref, bf_ref, out_ref):
    agg = agg_ref[0] + agg_ref[1]
    x3 = jnp.maximum(dis_ref[...] * agg + b_ref[...], 0.0)

    Pp = pp_ref[...]
    Pe = Pp[:, :64]
    Po = Pp[:, 64:]

    dn = (((0,), (0,)), ((), ()))

    def pool(xp):
        return (lax.dot_general(Pe, xp[:, :F], dn,
                                preferred_element_type=jnp.float32)
                + lax.dot_general(Po, xp[:, F:], dn,
                                  preferred_element_type=jnp.float32))

    s1 = pool(x1_ref[...])
    s2 = pool(x2_ref[...])
    s3 = pool(x3)
    pooled = jnp.concatenate([s1, s2, s3], axis=1)

    counts = jnp.sum(Pe, axis=0, keepdims=True) + jnp.sum(Po, axis=0,
                                                          keepdims=True)
    inv = 1.0 / jnp.maximum(counts, 1.0)
    pooled = pooled * inv.T

    logits = jnp.dot(pooled, wf_ref[...], preferred_element_type=jnp.float32)
    logits = logits + bf_ref[...]
    m = jnp.max(logits, axis=1, keepdims=True)
    e = jnp.exp(logits - m)
    out_ref[...] = e / jnp.sum(e, axis=1, keepdims=True)


def _tc_final(agg_p, dis2, bb3, x1p, x2p, Pp, Wf, bfr):
    return pl.pallas_call(
        _final_body,
        out_shape=jax.ShapeDtypeStruct((64, 10), jnp.float32),
    )(agg_p, dis2, bb3, x1p, x2p, Pp, Wf, bfr)


def _block_diag2(W):
    a, b = W.shape
    Z = jnp.zeros((a, b), W.dtype)
    return jnp.concatenate(
        [jnp.concatenate([W, Z], axis=1), jnp.concatenate([Z, W], axis=1)],
        axis=0)


def kernel(x, edge_index, batch, W1, b1, W2, b2, W3, b3, Wf, bf):
    n = x.shape[0]
    ei3 = edge_index.astype(jnp.int32).reshape(2, E_CHUNKS, CHUNK)

    x2 = x.reshape(NREAL2, 2 * x.shape[1])
    batch_pad = jnp.concatenate(
        [batch.astype(jnp.int32), jnp.full((N_NODES_P - n,), 64, jnp.int32)])
    Pp = (batch_pad[:, None] == jnp.arange(64, dtype=jnp.int32)[None, :]
          ).astype(jnp.float32).reshape(NP2, 128)

    deg_kernel = _make_degree_kernel()
    agg_kernel = _make_aggregate_kernel()

    dp = deg_kernel(ei3).reshape(NC, N_NODES_P * DEG_W // 128, 128)

    Wb1 = _block_diag2(W1)
    Wb2 = _block_diag2(W2)
    Wb3 = _block_diag2(W3)
    bb1 = jnp.concatenate([b1, b1]).reshape(1, 128)
    bb2 = jnp.concatenate([b2, b2]).reshape(1, 128)
    bb3 = jnp.concatenate([b3, b3]).reshape(1, 128)
    bfr = bf.reshape(1, 10)

    def agg(yp):
        parts = agg_kernel(ei3, yp.reshape(N_NODES_P, F))
        return parts.reshape(NC, NP2, 128)

    xw1 = _tc_xw1(x2, Wb1)
    y1p, dis2 = _tc_scale1(xw1, dp)
    x1p, y2p = _tc_mid(agg(y1p), dis2, bb1, Wb2)
    x2p, y3p = _tc_mid(agg(y2p), dis2, bb2, Wb3)
    return _tc_final(agg(y3p), dis2, bb3, x1p, x2p, Pp, Wf, bfr)
